# baseline (speedup 1.0000x reference)
import jax
import jax.numpy as jnp
from jax.experimental import pallas as pl
from jax.experimental.pallas import tpu as pltpu

B, C, H, W = 8, 3, 512, 512
D = 64
P = 8
G = 64
F = C * P * P


def _vq(zflat, cb):
    rown = jnp.sum(zflat * zflat, axis=-1, keepdims=True)
    cbn = jnp.sum(cb * cb, axis=-1)
    scores = jax.lax.dot_general(zflat, cb, (((1,), (1,)), ((), ())),
                                 preferred_element_type=jnp.float32)
    d = rown - 2.0 * scores + cbn[None, :]
    idx = jnp.argmin(d, axis=-1).astype(jnp.int32)
    onehot = (jax.lax.broadcasted_iota(jnp.int32, d.shape, 1)
              == idx[:, None]).astype(jnp.float32)
    zq = jnp.dot(onehot, cb, preferred_element_type=jnp.float32)
    return zq, idx


def _fused(x_ref, e1_ref, e1t_ref, wh_ref, wm_ref, wl_ref, cbh_ref, cbm_ref,
           cbl_ref, wd_ref, rec_ref, zh_ref, zm_ref, zl_ref, qh_ref, qm_ref,
           ql_ref, ih_ref, im_ref, il_ref):
    e1 = e1_ref[...]
    x2 = x_ref[0].reshape(C * H, W)
    xd = jnp.dot(x2, e1, preferred_element_type=jnp.float32,
                 precision=jax.lax.Precision.HIGHEST)
    tft = (xd.reshape(C, G, P, P, G)
             .transpose(0, 3, 2, 1, 4)
             .reshape(F, G * G))
    tf = tft.T
    t = tf.reshape(G, G, F)

    z_h = jnp.dot(tf, wh_ref[...], preferred_element_type=jnp.float32)

    t4 = t.reshape(32, 2, 32, 2, F)
    z_m = jnp.zeros((32 * 32, D), jnp.float32)
    for di in range(2):
        for dj in range(2):
            sub = t4[:, di, :, dj, :].reshape(32 * 32, F)
            z_m = z_m + jnp.dot(sub, wm_ref[2 * di + dj],
                                preferred_element_type=jnp.float32)

    t16 = t.reshape(16, 4, 16, 4, F)
    z_l = jnp.zeros((16 * 16, D), jnp.float32)
    for di in range(4):
        for dj in range(4):
            sub = t16[:, di, :, dj, :].reshape(16 * 16, F)
            z_l = z_l + jnp.dot(sub, wl_ref[4 * di + dj],
                                preferred_element_type=jnp.float32)

    q_h, i_h = _vq(z_h, cbh_ref[...])
    q_m, i_m = _vq(z_m, cbm_ref[...])
    q_l, i_l = _vq(z_l, cbl_ref[...])

    zh_ref[0] = z_h.reshape(G, G, D)
    zm_ref[0] = z_m.reshape(32, 32, D)
    zl_ref[0] = z_l.reshape(16, 16, D)
    qh_ref[0] = q_h.reshape(G, G, D)
    qm_ref[0] = q_m.reshape(32, 32, D)
    ql_ref[0] = q_l.reshape(16, 16, D)
    ih_ref[0] = i_h.reshape(G, G)
    im_ref[0] = i_m.reshape(32, 32)
    il_ref[0] = i_l.reshape(16, 16)

    up_m = jnp.broadcast_to(q_m.reshape(32, 1, 32, 1, D),
                            (32, 2, 32, 2, D)).reshape(G * G, D)
    up_l = jnp.broadcast_to(q_l.reshape(16, 1, 16, 1, D),
                            (16, 4, 16, 4, D)).reshape(G * G, D)
    h = q_h + up_m + up_l
    out = jnp.dot(h, wd_ref[...], preferred_element_type=jnp.float32)

    outt = out.T
    yd = (outt.reshape(C, P, P, G, G)
              .transpose(0, 3, 2, 1, 4)
              .reshape(C * H, W))
    rec = jnp.dot(yd, e1t_ref[...], preferred_element_type=jnp.float32,
                  precision=jax.lax.Precision.HIGHEST)
    rec_ref[0] = rec.reshape(C, H, W)


def kernel(x, W_enc_high, W_enc_mid, W_enc_low, cb_high, cb_mid, cb_low, W_dec):
    w_idx = jnp.arange(W)
    dest = (w_idx % P) * (W // P) + w_idx // P
    e1 = (dest[:, None] == jnp.arange(W)[None, :]).astype(jnp.float32)

    wh = W_enc_high.reshape(C, P, P, D).transpose(0, 2, 1, 3).reshape(F, D)
    wm = (W_enc_mid.reshape(C, 2, P, 2, P, D)
          .transpose(1, 3, 0, 4, 2, 5)
          .reshape(4, F, D))
    wl = (W_enc_low.reshape(C, 4, P, 4, P, D)
          .transpose(1, 3, 0, 4, 2, 5)
          .reshape(16, F, D))
    wd = W_dec.reshape(D, C, P, P).transpose(0, 1, 3, 2).reshape(D, F)

    full = lambda shape: pl.BlockSpec(shape, lambda b: (0,) * len(shape))
    outs = pl.pallas_call(
        _fused,
        grid=(B,),
        compiler_params=pltpu.CompilerParams(dimension_semantics=("parallel",),
                                             fuse_transposed_lhs_in_matmul=True),
        in_specs=[
            pl.BlockSpec((1, C, H, W), lambda b: (b, 0, 0, 0)),
            full((W, W)),
            full((W, W)),
            full((F, D)),
            full((4, F, D)),
            full((16, F, D)),
            full((256, D)),
            full((128, D)),
            full((128, D)),
            full((D, F)),
        ],
        out_specs=[
            pl.BlockSpec((1, C, H, W), lambda b: (b, 0, 0, 0)),
            pl.BlockSpec((1, G, G, D), lambda b: (b, 0, 0, 0)),
            pl.BlockSpec((1, 32, 32, D), lambda b: (b, 0, 0, 0)),
            pl.BlockSpec((1, 16, 16, D), lambda b: (b, 0, 0, 0)),
            pl.BlockSpec((1, G, G, D), lambda b: (b, 0, 0, 0)),
            pl.BlockSpec((1, 32, 32, D), lambda b: (b, 0, 0, 0)),
            pl.BlockSpec((1, 16, 16, D), lambda b: (b, 0, 0, 0)),
            pl.BlockSpec((1, G, G), lambda b: (b, 0, 0)),
            pl.BlockSpec((1, 32, 32), lambda b: (b, 0, 0)),
            pl.BlockSpec((1, 16, 16), lambda b: (b, 0, 0)),
        ],
        out_shape=[
            jax.ShapeDtypeStruct((B, C, H, W), jnp.float32),
            jax.ShapeDtypeStruct((B, G, G, D), jnp.float32),
            jax.ShapeDtypeStruct((B, 32, 32, D), jnp.float32),
            jax.ShapeDtypeStruct((B, 16, 16, D), jnp.float32),
            jax.ShapeDtypeStruct((B, G, G, D), jnp.float32),
            jax.ShapeDtypeStruct((B, 32, 32, D), jnp.float32),
            jax.ShapeDtypeStruct((B, 16, 16, D), jnp.float32),
            jax.ShapeDtypeStruct((B, G, G), jnp.int32),
            jax.ShapeDtypeStruct((B, 32, 32), jnp.int32),
            jax.ShapeDtypeStruct((B, 16, 16), jnp.int32),
        ],
    )(x, e1, e1.T, wh, wm, wl, cb_high, cb_mid, cb_low, wd)

    x_rec, z_h, z_m, z_l, q_h, q_m, q_l, i_h, i_m, i_l = outs
    return (x_rec, (z_h, z_m, z_l), (q_h, q_m, q_l), (i_h, i_m, i_l))


# speedup vs baseline: 1.4359x; 1.4359x over previous
import jax
import jax.numpy as jnp
from jax.experimental import pallas as pl
from jax.experimental.pallas import tpu as pltpu

B, C, H, W = 8, 3, 512, 512
D = 64
P = 8
G = 64
F = C * P * P


def _vq(zflat, cb):
    rown = jnp.sum(zflat * zflat, axis=-1, keepdims=True)
    cbn = jnp.sum(cb * cb, axis=-1)
    scores = jax.lax.dot_general(zflat, cb, (((1,), (1,)), ((), ())),
                                 preferred_element_type=jnp.float32)
    d = rown - 2.0 * scores + cbn[None, :]
    idx = jnp.argmin(d, axis=-1).astype(jnp.int32)
    onehot = (jax.lax.broadcasted_iota(jnp.int32, d.shape, 1)
              == idx[:, None]).astype(jnp.float32)
    zq = jnp.dot(onehot, cb, preferred_element_type=jnp.float32)
    return zq, idx


def _fused(x_ref, e1_ref, e1t_ref, wh_ref, wm_ref, wl_ref, cbh_ref, cbm_ref,
           cbl_ref, wd_ref, rec_ref, zh_ref, zm_ref, zl_ref, qh_ref, qm_ref,
           ql_ref, ih_ref, im_ref, il_ref):
    e1 = e1_ref[...]
    x2 = x_ref[0].reshape(C * H, W)
    xd = jnp.dot(x2, e1, preferred_element_type=jnp.float32,
                 precision=jax.lax.Precision.DEFAULT)
    tft = (xd.reshape(C, G, P, P, G)
             .transpose(0, 3, 2, 1, 4)
             .reshape(F, G * G))
    tf = tft.T
    t = tf.reshape(G, G, F)

    z_h = jnp.dot(tf, wh_ref[...], preferred_element_type=jnp.float32)

    t4 = t.reshape(32, 2, 32, 2, F)
    z_m = jnp.zeros((32 * 32, D), jnp.float32)
    for di in range(2):
        for dj in range(2):
            sub = t4[:, di, :, dj, :].reshape(32 * 32, F)
            z_m = z_m + jnp.dot(sub, wm_ref[2 * di + dj],
                                preferred_element_type=jnp.float32)

    t16 = t.reshape(16, 4, 16, 4, F)
    z_l = jnp.zeros((16 * 16, D), jnp.float32)
    for di in range(4):
        for dj in range(4):
            sub = t16[:, di, :, dj, :].reshape(16 * 16, F)
            z_l = z_l + jnp.dot(sub, wl_ref[4 * di + dj],
                                preferred_element_type=jnp.float32)

    q_h, i_h = _vq(z_h, cbh_ref[...])
    q_m, i_m = _vq(z_m, cbm_ref[...])
    q_l, i_l = _vq(z_l, cbl_ref[...])

    zh_ref[0] = z_h.reshape(G, G, D)
    zm_ref[0] = z_m.reshape(32, 32, D)
    zl_ref[0] = z_l.reshape(16, 16, D)
    qh_ref[0] = q_h.reshape(G, G, D)
    qm_ref[0] = q_m.reshape(32, 32, D)
    ql_ref[0] = q_l.reshape(16, 16, D)
    ih_ref[0] = i_h.reshape(G, G)
    im_ref[0] = i_m.reshape(32, 32)
    il_ref[0] = i_l.reshape(16, 16)

    up_m = jnp.broadcast_to(q_m.reshape(32, 1, 32, 1, D),
                            (32, 2, 32, 2, D)).reshape(G * G, D)
    up_l = jnp.broadcast_to(q_l.reshape(16, 1, 16, 1, D),
                            (16, 4, 16, 4, D)).reshape(G * G, D)
    h = q_h + up_m + up_l
    out = jnp.dot(h, wd_ref[...], preferred_element_type=jnp.float32)

    outt = out.T
    yd = (outt.reshape(C, P, P, G, G)
              .transpose(0, 3, 2, 1, 4)
              .reshape(C * H, W))
    rec = jnp.dot(yd, e1t_ref[...], preferred_element_type=jnp.float32,
                  precision=jax.lax.Precision.DEFAULT)
    rec_ref[0] = rec.reshape(C, H, W)


def kernel(x, W_enc_high, W_enc_mid, W_enc_low, cb_high, cb_mid, cb_low, W_dec):
    w_idx = jnp.arange(W)
    dest = (w_idx % P) * (W // P) + w_idx // P
    e1 = (dest[:, None] == jnp.arange(W)[None, :]).astype(jnp.float32)

    wh = W_enc_high.reshape(C, P, P, D).transpose(0, 2, 1, 3).reshape(F, D)
    wm = (W_enc_mid.reshape(C, 2, P, 2, P, D)
          .transpose(1, 3, 0, 4, 2, 5)
          .reshape(4, F, D))
    wl = (W_enc_low.reshape(C, 4, P, 4, P, D)
          .transpose(1, 3, 0, 4, 2, 5)
          .reshape(16, F, D))
    wd = W_dec.reshape(D, C, P, P).transpose(0, 1, 3, 2).reshape(D, F)

    full = lambda shape: pl.BlockSpec(shape, lambda b: (0,) * len(shape))
    outs = pl.pallas_call(
        _fused,
        grid=(B,),
        compiler_params=pltpu.CompilerParams(dimension_semantics=("parallel",),
                                             fuse_transposed_lhs_in_matmul=True),
        in_specs=[
            pl.BlockSpec((1, C, H, W), lambda b: (b, 0, 0, 0)),
            full((W, W)),
            full((W, W)),
            full((F, D)),
            full((4, F, D)),
            full((16, F, D)),
            full((256, D)),
            full((128, D)),
            full((128, D)),
            full((D, F)),
        ],
        out_specs=[
            pl.BlockSpec((1, C, H, W), lambda b: (b, 0, 0, 0)),
            pl.BlockSpec((1, G, G, D), lambda b: (b, 0, 0, 0)),
            pl.BlockSpec((1, 32, 32, D), lambda b: (b, 0, 0, 0)),
            pl.BlockSpec((1, 16, 16, D), lambda b: (b, 0, 0, 0)),
            pl.BlockSpec((1, G, G, D), lambda b: (b, 0, 0, 0)),
            pl.BlockSpec((1, 32, 32, D), lambda b: (b, 0, 0, 0)),
            pl.BlockSpec((1, 16, 16, D), lambda b: (b, 0, 0, 0)),
            pl.BlockSpec((1, G, G), lambda b: (b, 0, 0)),
            pl.BlockSpec((1, 32, 32), lambda b: (b, 0, 0)),
            pl.BlockSpec((1, 16, 16), lambda b: (b, 0, 0)),
        ],
        out_shape=[
            jax.ShapeDtypeStruct((B, C, H, W), jnp.float32),
            jax.ShapeDtypeStruct((B, G, G, D), jnp.float32),
            jax.ShapeDtypeStruct((B, 32, 32, D), jnp.float32),
            jax.ShapeDtypeStruct((B, 16, 16, D), jnp.float32),
            jax.ShapeDtypeStruct((B, G, G, D), jnp.float32),
            jax.ShapeDtypeStruct((B, 32, 32, D), jnp.float32),
            jax.ShapeDtypeStruct((B, 16, 16, D), jnp.float32),
            jax.ShapeDtypeStruct((B, G, G), jnp.int32),
            jax.ShapeDtypeStruct((B, 32, 32), jnp.int32),
            jax.ShapeDtypeStruct((B, 16, 16), jnp.int32),
        ],
    )(x, e1, e1.T, wh, wm, wl, cb_high, cb_mid, cb_low, wd)

    x_rec, z_h, z_m, z_l, q_h, q_m, q_l, i_h, i_m, i_l = outs
    return (x_rec, (z_h, z_m, z_l), (q_h, q_m, q_l), (i_h, i_m, i_l))
